# baseline (speedup 1.0000x reference)
import functools

import jax
import jax.numpy as jnp
from jax import lax
from jax.experimental import pallas as pl
from jax.experimental.pallas import tpu as pltpu
from jax.experimental.pallas import tpu_sc as plsc

N = 10000
E = 320000
D = 128
NC = 2
NS = 16
NW = NC * NS
CH = 128
NCHUNK = 80
EP = NW * NCHUNK * CH
G = 16
NG = NCHUNK // G
HS = 2
CHH = CH // HS
RPT = 624
TCB = 1000

_f32 = jnp.float32
_i32 = jnp.int32

_MESH = plsc.VectorSubcoreMesh(
    core_axis_name="c", subcore_axis_name="s", num_cores=NC, num_subcores=NS)


@functools.partial(
    pl.kernel,
    out_type=jax.ShapeDtypeStruct((NC, N, D), _f32),
    mesh=_MESH,
    scratch_types=[
        pltpu.VMEM_SHARED((N, D), _f32),
        pltpu.VMEM((3, 2, HS, CHH), _i32),
        pltpu.VMEM((3, CH), _f32),
        pltpu.VMEM((2, CH, D), _f32),
        pltpu.SemaphoreType.DMA,
        pltpu.SemaphoreType.DMA,
        pltpu.SemaphoreType.DMA,
        pltpu.SemaphoreType.DMA,
    ],
)
def _spmm_sc(table, idxr, wr, out, accum, idx_v, w_v, rows,
             isem, wsem, gsem, ssem):
    c = lax.axis_index("c")
    s = lax.axis_index("s")
    wid = s * NC + c

    z16 = jnp.zeros((16,), _f32)

    @pl.loop(0, CH)
    def _zero_rows(i):
        for g in range(D // 16):
            rows[0, i, pl.ds(g * 16, 16)] = z16

    base = s * RPT
    for k in range(RPT // CH):
        pltpu.sync_copy(rows.at[0], accum.at[pl.ds(base + k * CH, CH)])

    @pl.when(s == NS - 1)
    def _zero_tail_full():
        pltpu.sync_copy(rows.at[0], accum.at[pl.ds(base + 4 * CH, CH)])

    @pl.when(s != NS - 1)
    def _zero_tail_part():
        pltpu.sync_copy(rows.at[0, pl.ds(0, RPT - 4 * CH)],
                        accum.at[pl.ds(base + 4 * CH, RPT - 4 * CH)])

    plsc.subcore_barrier()

    pltpu.sync_copy(idxr.at[wid, 0], idx_v.at[0])
    pltpu.sync_copy(wr.at[wid, 0], w_v.at[0])
    for hh in range(HS):
        pltpu.async_copy(table.at[idx_v.at[0, 0, hh]],
                         rows.at[0, pl.ds(hh * CHH, CHH)], gsem)
    pltpu.async_copy(idxr.at[wid, 1], idx_v.at[1], isem)
    pltpu.async_copy(wr.at[wid, 1], w_v.at[1], wsem)

    @pl.loop(0, NCHUNK)
    def _chunk(j):
        p = j % 2
        pn = (j + 1) % 2
        b0 = j % 3
        b1 = (j + 1) % 3
        b2 = (j + 2) % 3

        @pl.when(j >= 1)
        def _drain_prev_scatter():
            for hh in range(HS):
                pltpu.make_async_copy(
                    rows.at[pn, pl.ds(hh * CHH, CHH)],
                    accum.at[idx_v.at[b2, 1, hh]], ssem).wait()

        @pl.when(j + 1 < NCHUNK)
        def _start_next_gather():
            pltpu.make_async_copy(idxr.at[wid, j + 1], idx_v.at[b1],
                                  isem).wait()
            for hh in range(HS):
                pltpu.async_copy(
                    table.at[idx_v.at[b1, 0, hh]],
                    rows.at[pn, pl.ds(hh * CHH, CHH)], gsem)

        @pl.when(j + 2 < NCHUNK)
        def _start_next_idx():
            pltpu.async_copy(idxr.at[wid, j + 2], idx_v.at[b2], isem)
            pltpu.async_copy(wr.at[wid, j + 2], w_v.at[b2], wsem)

        for hh in range(HS):
            pltpu.make_async_copy(
                table.at[idx_v.at[b0, 0, hh]],
                rows.at[p, pl.ds(hh * CHH, CHH)], gsem).wait()

        @pl.when(j >= 1)
        def _wait_w():
            pltpu.make_async_copy(wr.at[wid, j], w_v.at[b0], wsem).wait()

        @pl.loop(0, CH // 16)
        def _scale(t):
            wg = w_v[b0, pl.ds(t * 16, 16)]
            for k in range(16):
                e = t * 16 + k
                wv = jnp.full((16,), wg[k], _f32)
                for g in range(D // 16):
                    rows[p, e, pl.ds(g * 16, 16)] = (
                        rows[p, e, pl.ds(g * 16, 16)] * wv)

        for hh in range(HS):
            pltpu.async_copy(rows.at[p, pl.ds(hh * CHH, CHH)],
                             accum.at[idx_v.at[b0, 1, hh]], ssem,
                             add=True)

    for hh in range(HS):
        pltpu.make_async_copy(
            rows.at[(NCHUNK - 1) % 2, pl.ds(hh * CHH, CHH)],
            accum.at[idx_v.at[(NCHUNK - 1) % 3, 1, hh]], ssem).wait()

    plsc.subcore_barrier()
    for k in range(RPT // CH):
        pltpu.sync_copy(accum.at[pl.ds(base + k * CH, CH)],
                        out.at[c, pl.ds(base + k * CH, CH)])

    @pl.when(s == NS - 1)
    def _write_tail_full():
        pltpu.sync_copy(accum.at[pl.ds(base + 4 * CH, CH)],
                        out.at[c, pl.ds(base + 4 * CH, CH)])

    @pl.when(s != NS - 1)
    def _write_tail_part():
        pltpu.sync_copy(accum.at[pl.ds(base + 4 * CH, RPT - 4 * CH)],
                        out.at[c, pl.ds(base + 4 * CH, RPT - 4 * CH)])


@functools.partial(
    pl.kernel,
    out_type=jax.ShapeDtypeStruct((NW, NCHUNK, CH, 48), _f32),
    mesh=_MESH,
    scratch_types=[
        pltpu.VMEM((3, 2, HS, CHH), _i32),
        pltpu.VMEM((2, CH, D), _f32),
        pltpu.VMEM((2, CH, D), _f32),
        pltpu.VMEM((2, CH, 48), _f32),
        pltpu.SemaphoreType.DMA,
        pltpu.SemaphoreType.DMA,
        pltpu.SemaphoreType.DMA,
    ],
)
def _attn_sc(gtab, idxr, part_o, idx_v, arows, brows, pall,
             isem, gsem, osem):
    c = lax.axis_index("c")
    s = lax.axis_index("s")
    wid = s * NC + c

    pltpu.sync_copy(idxr.at[wid, 0], idx_v.at[0])
    for hh in range(HS):
        pltpu.async_copy(gtab.at[idx_v.at[0, 0, hh]],
                         arows.at[0, pl.ds(hh * CHH, CHH)], gsem)
        pltpu.async_copy(gtab.at[idx_v.at[0, 1, hh]],
                         brows.at[0, pl.ds(hh * CHH, CHH)], gsem)
    pltpu.async_copy(idxr.at[wid, 1], idx_v.at[1], isem)

    @pl.loop(0, NCHUNK)
    def _chunk(j):
        p = j % 2
        pn = (j + 1) % 2
        b0 = j % 3
        b1 = (j + 1) % 3
        b2 = (j + 2) % 3

        @pl.when(j >= 1)
        def _drain_prev_out():
            pltpu.make_async_copy(pall.at[pn], part_o.at[wid, j - 1],
                                  osem).wait()

        @pl.when(j + 1 < NCHUNK)
        def _start_next_gather():
            pltpu.make_async_copy(idxr.at[wid, j + 1], idx_v.at[b1],
                                  isem).wait()
            for hh in range(HS):
                pltpu.async_copy(gtab.at[idx_v.at[b1, 0, hh]],
                                 arows.at[pn, pl.ds(hh * CHH, CHH)], gsem)
                pltpu.async_copy(gtab.at[idx_v.at[b1, 1, hh]],
                                 brows.at[pn, pl.ds(hh * CHH, CHH)], gsem)

        @pl.when(j + 2 < NCHUNK)
        def _start_next_idx():
            pltpu.async_copy(idxr.at[wid, j + 2], idx_v.at[b2], isem)

        for hh in range(HS):
            pltpu.make_async_copy(gtab.at[idx_v.at[b0, 0, hh]],
                                  arows.at[p, pl.ds(hh * CHH, CHH)],
                                  gsem).wait()
            pltpu.make_async_copy(gtab.at[idx_v.at[b0, 1, hh]],
                                  brows.at[p, pl.ds(hh * CHH, CHH)],
                                  gsem).wait()

        @pl.loop(0, CH, unroll=2)
        def _edge(e):
            a = arows[p, e, pl.ds(0, 16)]
            b = brows[p, e, pl.ds(0, 16)]
            pab = a * b
            paa = a * a
            pbb = b * b
            for g in range(1, D // 16):
                a = arows[p, e, pl.ds(g * 16, 16)]
                b = brows[p, e, pl.ds(g * 16, 16)]
                pab = pab + a * b
                paa = paa + a * a
                pbb = pbb + b * b
            pall[p, e, pl.ds(0, 16)] = pab
            pall[p, e, pl.ds(16, 16)] = paa
            pall[p, e, pl.ds(32, 16)] = pbb

        pltpu.async_copy(pall.at[p], part_o.at[wid, j], osem)

    pltpu.make_async_copy(pall.at[(NCHUNK - 1) % 2],
                          part_o.at[wid, NCHUNK - 1], osem).wait()


def _dotT(a, w):
    return lax.dot_general(a, w, (((1,), (1,)), ((), ())),
                           preferred_element_type=_f32)


def _tc_in(x, W_in, b_in, W1, b1):
    def body(x_r, wi_r, bi_r, w1_r, b1_r, h_r, lin1_r):
        h = _dotT(x_r[...], wi_r[...]) + bi_r[...][None, :]
        h_r[...] = h
        lin1_r[...] = _dotT(h, w1_r[...]) + b1_r[...][None, :]

    blk = pl.BlockSpec((TCB, D), lambda i: (i, 0))
    wspec = pl.BlockSpec((D, D), lambda i: (0, 0))
    bspec = pl.BlockSpec((D,), lambda i: (0,))
    return pl.pallas_call(
        body,
        grid=(N // TCB,),
        in_specs=[blk, wspec, bspec, wspec, bspec],
        out_specs=[blk, blk],
        out_shape=[jax.ShapeDtypeStruct((N, D), _f32)] * 2,
    )(x, W_in, b_in, W1, b1)


def _tc_layer(p, h, W2, b2, temp):
    def body(p_r, h_r, w2_r, b2_r, t_r, lin2_r, hidp_r):
        cur = jnp.maximum(p_r[0] + p_r[1], 0.0)
        lin2_r[...] = _dotT(cur, w2_r[...]) + b2_r[...][None, :]
        hidp_r[...] = h_r[...] * t_r[0] + cur * t_r[1]

    blk = pl.BlockSpec((TCB, D), lambda i: (i, 0))
    pblk = pl.BlockSpec((NC, TCB, D), lambda i: (0, i, 0))
    wspec = pl.BlockSpec((D, D), lambda i: (0, 0))
    bspec = pl.BlockSpec((D,), lambda i: (0,))
    tspec = pl.BlockSpec(memory_space=pltpu.SMEM)
    return pl.pallas_call(
        body,
        grid=(N // TCB,),
        in_specs=[pblk, blk, wspec, bspec, tspec],
        out_specs=[blk, blk],
        out_shape=[jax.ShapeDtypeStruct((N, D), _f32)] * 2,
    )(p, h, W2, b2, temp)


def _tc_extract(p2, hidp, temp, We1, be1, We2, be2):
    def body(p_r, hidp_r, t_r, we1_r, be1_r, we2_r, be2_r, g_r):
        cur2 = jnp.maximum(p_r[0] + p_r[1], 0.0)
        hgnn = hidp_r[...] + cur2 * t_r[2]
        t1 = jnp.maximum(_dotT(hgnn, we1_r[...]) + be1_r[...][None, :], 0.0)
        g_r[...] = _dotT(t1, we2_r[...]) + be2_r[...][None, :]

    blk = pl.BlockSpec((TCB, D), lambda i: (i, 0))
    pblk = pl.BlockSpec((NC, TCB, D), lambda i: (0, i, 0))
    wspec = pl.BlockSpec((D, D), lambda i: (0, 0))
    bspec = pl.BlockSpec((D,), lambda i: (0,))
    tspec = pl.BlockSpec(memory_space=pltpu.SMEM)
    return pl.pallas_call(
        body,
        grid=(N // TCB,),
        in_specs=[pblk, blk, tspec, wspec, bspec, wspec, bspec],
        out_specs=blk,
        out_shape=jax.ShapeDtypeStruct((N, D), _f32),
    )(p2, hidp, temp, We1, be1, We2, be2)


def _tc_attnw(wf, part):
    TB = 8192

    def body(w_r, p_r, o_r):
        r = lax.broadcasted_iota(_i32, (48, 3), 0)
        cc = lax.broadcasted_iota(_i32, (48, 3), 1)
        sel = ((r // 16) == cc).astype(_f32)
        sums = jnp.dot(p_r[...], sel, preferred_element_type=_f32)
        num = sums[:, 0]
        den = jnp.maximum(jnp.sqrt(sums[:, 1] * sums[:, 2]), 1e-8)
        o_r[...] = w_r[...] * (num / den)

    return pl.pallas_call(
        body,
        grid=(EP // TB,),
        in_specs=[pl.BlockSpec((TB,), lambda i: (i,)),
                  pl.BlockSpec((TB, 48), lambda i: (i, 0))],
        out_specs=pl.BlockSpec((TB,), lambda i: (i,)),
        out_shape=jax.ShapeDtypeStruct((EP,), _f32),
    )(wf, part)


def _tc_out(p4, hidp2, temp, W_out, b_out):
    def body(p_r, hidp_r, t_r, wo_r, bo_r, o_r):
        cur = jnp.maximum(p_r[0] + p_r[1], 0.0)
        hgnn2 = hidp_r[...] + cur * t_r[2]
        o_r[...] = _dotT(hgnn2, wo_r[...]) + bo_r[...][None, :]

    blk = pl.BlockSpec((TCB, D), lambda i: (i, 0))
    pblk = pl.BlockSpec((NC, TCB, D), lambda i: (0, i, 0))
    wspec = pl.BlockSpec((D, D), lambda i: (0, 0))
    bspec = pl.BlockSpec((D,), lambda i: (0,))
    tspec = pl.BlockSpec(memory_space=pltpu.SMEM)
    return pl.pallas_call(
        body,
        grid=(N // TCB,),
        in_specs=[pblk, blk, tspec, wspec, bspec],
        out_specs=blk,
        out_shape=jax.ShapeDtypeStruct((N, D), _f32),
    )(p4, hidp2, temp, W_out, b_out)


def kernel(x, edge_index, edge_w, W_in, b_in, W1, b1, W2, b2,
           We1, be1, We2, be2, W_out, b_out, temp):
    padi = jnp.zeros((EP - E,), _i32)
    src_r = jnp.concatenate([edge_index[0], padi]).reshape(NW, NCHUNK, CH)
    dst_r = jnp.concatenate([edge_index[1], padi]).reshape(NW, NCHUNK, CH)
    idxr = jnp.stack([src_r, dst_r], axis=2).reshape(
        NW, NCHUNK, 2, HS, CHH)
    padf = jnp.zeros((EP - E,), _f32)
    w_r = jnp.concatenate([edge_w, padf]).reshape(NW, NCHUNK, CH)

    h, lin1 = _tc_in(x, W_in, b_in, W1, b1)
    p1 = _spmm_sc(lin1, idxr, w_r)
    lin2, hidp = _tc_layer(p1, h, W2, b2, temp)
    p2 = _spmm_sc(lin2, idxr, w_r)
    g = _tc_extract(p2, hidp, temp, We1, be1, We2, be2)
    part = _attn_sc(g, idxr)
    w2 = _tc_attnw(w_r.reshape(EP), part.reshape(EP, 48))
    w2_r = w2.reshape(NW, NCHUNK, CH)

    p3 = _spmm_sc(lin1, idxr, w2_r)
    lin2b, hidp2 = _tc_layer(p3, h, W2, b2, temp)
    p4 = _spmm_sc(lin2b, idxr, w2_r)
    return _tc_out(p4, hidp2, temp, W_out, b_out)


# speedup vs baseline: 1.2520x; 1.2520x over previous
import functools

import jax
import jax.numpy as jnp
from jax import lax
from jax.experimental import pallas as pl
from jax.experimental.pallas import tpu as pltpu
from jax.experimental.pallas import tpu_sc as plsc

N = 10000
E = 320000
D = 128
NC = 2
NS = 16
NW = NC * NS
CH = 128
NCHUNK = 80
EP = NW * NCHUNK * CH
G = 16
NG = NCHUNK // G
HS = 2
CHH = CH // HS
RPT = 624
TCB = 1000

_f32 = jnp.float32
_i32 = jnp.int32

_MESH = plsc.VectorSubcoreMesh(
    core_axis_name="c", subcore_axis_name="s", num_cores=NC, num_subcores=NS)


@functools.partial(
    pl.kernel,
    out_type=jax.ShapeDtypeStruct((NC, N, D), _f32),
    mesh=_MESH,
    scratch_types=[
        pltpu.VMEM_SHARED((N, D), _f32),
        pltpu.VMEM((3, 2, HS, CHH), _i32),
        pltpu.VMEM((3, CH), _f32),
        pltpu.VMEM((2, CH, D), _f32),
        pltpu.SemaphoreType.DMA,
        pltpu.SemaphoreType.DMA,
        pltpu.SemaphoreType.DMA,
        pltpu.SemaphoreType.DMA,
    ],
)
def _spmm_sc(table, idxr, wr, out, accum, idx_v, w_v, rows,
             isem, wsem, gsem, ssem):
    c = lax.axis_index("c")
    s = lax.axis_index("s")
    wid = s * NC + c

    z16 = jnp.zeros((16,), _f32)

    @pl.loop(0, CH)
    def _zero_rows(i):
        for g in range(D // 16):
            rows[0, i, pl.ds(g * 16, 16)] = z16

    base = s * RPT
    for k in range(RPT // CH):
        pltpu.sync_copy(rows.at[0], accum.at[pl.ds(base + k * CH, CH)])

    @pl.when(s == NS - 1)
    def _zero_tail_full():
        pltpu.sync_copy(rows.at[0], accum.at[pl.ds(base + 4 * CH, CH)])

    @pl.when(s != NS - 1)
    def _zero_tail_part():
        pltpu.sync_copy(rows.at[0, pl.ds(0, RPT - 4 * CH)],
                        accum.at[pl.ds(base + 4 * CH, RPT - 4 * CH)])

    plsc.subcore_barrier()

    pltpu.sync_copy(idxr.at[wid, 0], idx_v.at[0])
    pltpu.sync_copy(wr.at[wid, 0], w_v.at[0])
    for hh in range(HS):
        pltpu.async_copy(table.at[idx_v.at[0, 0, hh]],
                         rows.at[0, pl.ds(hh * CHH, CHH)], gsem)
    pltpu.async_copy(idxr.at[wid, 1], idx_v.at[1], isem)
    pltpu.async_copy(wr.at[wid, 1], w_v.at[1], wsem)

    @pl.loop(0, NCHUNK)
    def _chunk(j):
        p = j % 2
        pn = (j + 1) % 2
        b0 = j % 3
        b1 = (j + 1) % 3
        b2 = (j + 2) % 3

        @pl.when(j >= 1)
        def _drain_prev_scatter():
            for hh in range(HS):
                pltpu.make_async_copy(
                    rows.at[pn, pl.ds(hh * CHH, CHH)],
                    accum.at[idx_v.at[b2, 1, hh]], ssem).wait()

        @pl.when(j + 1 < NCHUNK)
        def _start_next_gather():
            pltpu.make_async_copy(idxr.at[wid, j + 1], idx_v.at[b1],
                                  isem).wait()
            for hh in range(HS):
                pltpu.async_copy(
                    table.at[idx_v.at[b1, 0, hh]],
                    rows.at[pn, pl.ds(hh * CHH, CHH)], gsem)

        @pl.when(j + 2 < NCHUNK)
        def _start_next_idx():
            pltpu.async_copy(idxr.at[wid, j + 2], idx_v.at[b2], isem)
            pltpu.async_copy(wr.at[wid, j + 2], w_v.at[b2], wsem)

        for hh in range(HS):
            pltpu.make_async_copy(
                table.at[idx_v.at[b0, 0, hh]],
                rows.at[p, pl.ds(hh * CHH, CHH)], gsem).wait()

        @pl.when(j >= 1)
        def _wait_w():
            pltpu.make_async_copy(wr.at[wid, j], w_v.at[b0], wsem).wait()

        @pl.loop(0, CH // 16)
        def _scale(t):
            wg = w_v[b0, pl.ds(t * 16, 16)]
            for k in range(16):
                e = t * 16 + k
                wv = jnp.full((16,), wg[k], _f32)
                for g in range(D // 16):
                    rows[p, e, pl.ds(g * 16, 16)] = (
                        rows[p, e, pl.ds(g * 16, 16)] * wv)

        for hh in range(HS):
            pltpu.async_copy(rows.at[p, pl.ds(hh * CHH, CHH)],
                             accum.at[idx_v.at[b0, 1, hh]], ssem,
                             add=True)

    for hh in range(HS):
        pltpu.make_async_copy(
            rows.at[(NCHUNK - 1) % 2, pl.ds(hh * CHH, CHH)],
            accum.at[idx_v.at[(NCHUNK - 1) % 3, 1, hh]], ssem).wait()

    plsc.subcore_barrier()
    for k in range(RPT // CH):
        pltpu.sync_copy(accum.at[pl.ds(base + k * CH, CH)],
                        out.at[c, pl.ds(base + k * CH, CH)])

    @pl.when(s == NS - 1)
    def _write_tail_full():
        pltpu.sync_copy(accum.at[pl.ds(base + 4 * CH, CH)],
                        out.at[c, pl.ds(base + 4 * CH, CH)])

    @pl.when(s != NS - 1)
    def _write_tail_part():
        pltpu.sync_copy(accum.at[pl.ds(base + 4 * CH, RPT - 4 * CH)],
                        out.at[c, pl.ds(base + 4 * CH, RPT - 4 * CH)])


ACH = 64
ANCH = (EP // NW) // ACH
APACK = ACH * 48


@functools.partial(
    pl.kernel,
    out_type=jax.ShapeDtypeStruct((NW, ANCH, APACK), _f32),
    mesh=_MESH,
    scratch_types=[
        pltpu.VMEM_SHARED((N, D), _f32),
        pltpu.VMEM((3, 2, ACH), _i32),
        pltpu.VMEM((2, ACH, D), _f32),
        pltpu.VMEM((2, ACH, D), _f32),
        pltpu.VMEM((2, APACK), _f32),
        pltpu.SemaphoreType.DMA,
        pltpu.SemaphoreType.DMA,
        pltpu.SemaphoreType.DMA,
    ],
)
def _attn_sc(gtab, idxr, part_o, stab, idx_v, arows, brows, pall,
             isem, gsem, osem):
    c = lax.axis_index("c")
    s = lax.axis_index("s")
    wid = s * NC + c

    base = s * RPT
    pltpu.sync_copy(gtab.at[pl.ds(base, RPT)], stab.at[pl.ds(base, RPT)])

    @pl.when(s == NS - 1)
    def _stage_tail():
        pltpu.sync_copy(gtab.at[pl.ds(N - 16, 16)],
                        stab.at[pl.ds(N - 16, 16)])

    plsc.subcore_barrier()

    pltpu.sync_copy(idxr.at[wid, 0], idx_v.at[0])
    pltpu.async_copy(stab.at[idx_v.at[0, 0]], arows.at[0], gsem)
    pltpu.async_copy(stab.at[idx_v.at[0, 1]], brows.at[0], gsem)
    pltpu.async_copy(idxr.at[wid, 1], idx_v.at[1], isem)

    @pl.loop(0, ANCH)
    def _chunk(j):
        p = j % 2
        pn = (j + 1) % 2
        b0 = j % 3
        b1 = (j + 1) % 3
        b2 = (j + 2) % 3

        @pl.when(j >= 1)
        def _drain_prev_out():
            pltpu.make_async_copy(pall.at[pn], part_o.at[wid, j - 1],
                                  osem).wait()

        @pl.when(j + 1 < ANCH)
        def _start_next_gather():
            pltpu.make_async_copy(idxr.at[wid, j + 1], idx_v.at[b1],
                                  isem).wait()
            pltpu.async_copy(stab.at[idx_v.at[b1, 0]], arows.at[pn], gsem)
            pltpu.async_copy(stab.at[idx_v.at[b1, 1]], brows.at[pn], gsem)

        @pl.when(j + 2 < ANCH)
        def _start_next_idx():
            pltpu.async_copy(idxr.at[wid, j + 2], idx_v.at[b2], isem)

        pltpu.make_async_copy(stab.at[idx_v.at[b0, 0]], arows.at[p],
                              gsem).wait()
        pltpu.make_async_copy(stab.at[idx_v.at[b0, 1]], brows.at[p],
                              gsem).wait()

        @pl.loop(0, ACH, unroll=2)
        def _edge(e):
            a = arows[p, e, pl.ds(0, 16)]
            b = brows[p, e, pl.ds(0, 16)]
            pab = a * b
            paa = a * a
            pbb = b * b
            for g in range(1, D // 16):
                a = arows[p, e, pl.ds(g * 16, 16)]
                b = brows[p, e, pl.ds(g * 16, 16)]
                pab = pab + a * b
                paa = paa + a * a
                pbb = pbb + b * b
            pall[p, pl.ds(e * 48, 16)] = pab
            pall[p, pl.ds(e * 48 + 16, 16)] = paa
            pall[p, pl.ds(e * 48 + 32, 16)] = pbb

        pltpu.async_copy(pall.at[p], part_o.at[wid, j], osem)

    pltpu.make_async_copy(pall.at[(ANCH - 1) % 2],
                          part_o.at[wid, ANCH - 1], osem).wait()


def _dotT(a, w):
    return lax.dot_general(a, w, (((1,), (1,)), ((), ())),
                           preferred_element_type=_f32)


def _tc_in(x, W_in, b_in, W1, b1):
    def body(x_r, wi_r, bi_r, w1_r, b1_r, h_r, lin1_r):
        h = _dotT(x_r[...], wi_r[...]) + bi_r[...][None, :]
        h_r[...] = h
        lin1_r[...] = _dotT(h, w1_r[...]) + b1_r[...][None, :]

    blk = pl.BlockSpec((TCB, D), lambda i: (i, 0))
    wspec = pl.BlockSpec((D, D), lambda i: (0, 0))
    bspec = pl.BlockSpec((D,), lambda i: (0,))
    return pl.pallas_call(
        body,
        grid=(N // TCB,),
        in_specs=[blk, wspec, bspec, wspec, bspec],
        out_specs=[blk, blk],
        out_shape=[jax.ShapeDtypeStruct((N, D), _f32)] * 2,
    )(x, W_in, b_in, W1, b1)


def _tc_layer(p, h, W2, b2, temp):
    def body(p_r, h_r, w2_r, b2_r, t_r, lin2_r, hidp_r):
        cur = jnp.maximum(p_r[0] + p_r[1], 0.0)
        lin2_r[...] = _dotT(cur, w2_r[...]) + b2_r[...][None, :]
        hidp_r[...] = h_r[...] * t_r[0] + cur * t_r[1]

    blk = pl.BlockSpec((TCB, D), lambda i: (i, 0))
    pblk = pl.BlockSpec((NC, TCB, D), lambda i: (0, i, 0))
    wspec = pl.BlockSpec((D, D), lambda i: (0, 0))
    bspec = pl.BlockSpec((D,), lambda i: (0,))
    tspec = pl.BlockSpec(memory_space=pltpu.SMEM)
    return pl.pallas_call(
        body,
        grid=(N // TCB,),
        in_specs=[pblk, blk, wspec, bspec, tspec],
        out_specs=[blk, blk],
        out_shape=[jax.ShapeDtypeStruct((N, D), _f32)] * 2,
    )(p, h, W2, b2, temp)


def _tc_extract(p2, hidp, temp, We1, be1, We2, be2):
    def body(p_r, hidp_r, t_r, we1_r, be1_r, we2_r, be2_r, g_r):
        cur2 = jnp.maximum(p_r[0] + p_r[1], 0.0)
        hgnn = hidp_r[...] + cur2 * t_r[2]
        t1 = jnp.maximum(_dotT(hgnn, we1_r[...]) + be1_r[...][None, :], 0.0)
        g_r[...] = _dotT(t1, we2_r[...]) + be2_r[...][None, :]

    blk = pl.BlockSpec((TCB, D), lambda i: (i, 0))
    pblk = pl.BlockSpec((NC, TCB, D), lambda i: (0, i, 0))
    wspec = pl.BlockSpec((D, D), lambda i: (0, 0))
    bspec = pl.BlockSpec((D,), lambda i: (0,))
    tspec = pl.BlockSpec(memory_space=pltpu.SMEM)
    return pl.pallas_call(
        body,
        grid=(N // TCB,),
        in_specs=[pblk, blk, tspec, wspec, bspec, wspec, bspec],
        out_specs=blk,
        out_shape=jax.ShapeDtypeStruct((N, D), _f32),
    )(p2, hidp, temp, We1, be1, We2, be2)


def _tc_attnw(wf, part):
    TB = 8192

    def body(w_r, p_r, o_r):
        r = lax.broadcasted_iota(_i32, (48, 3), 0)
        cc = lax.broadcasted_iota(_i32, (48, 3), 1)
        sel = ((r // 16) == cc).astype(_f32)
        sums = jnp.dot(p_r[...], sel, preferred_element_type=_f32)
        num = sums[:, 0]
        den = jnp.maximum(jnp.sqrt(sums[:, 1] * sums[:, 2]), 1e-8)
        o_r[...] = w_r[...] * (num / den)

    return pl.pallas_call(
        body,
        grid=(EP // TB,),
        in_specs=[pl.BlockSpec((TB,), lambda i: (i,)),
                  pl.BlockSpec((TB, 48), lambda i: (i, 0))],
        out_specs=pl.BlockSpec((TB,), lambda i: (i,)),
        out_shape=jax.ShapeDtypeStruct((EP,), _f32),
    )(wf, part)


def _tc_out(p4, hidp2, temp, W_out, b_out):
    def body(p_r, hidp_r, t_r, wo_r, bo_r, o_r):
        cur = jnp.maximum(p_r[0] + p_r[1], 0.0)
        hgnn2 = hidp_r[...] + cur * t_r[2]
        o_r[...] = _dotT(hgnn2, wo_r[...]) + bo_r[...][None, :]

    blk = pl.BlockSpec((TCB, D), lambda i: (i, 0))
    pblk = pl.BlockSpec((NC, TCB, D), lambda i: (0, i, 0))
    wspec = pl.BlockSpec((D, D), lambda i: (0, 0))
    bspec = pl.BlockSpec((D,), lambda i: (0,))
    tspec = pl.BlockSpec(memory_space=pltpu.SMEM)
    return pl.pallas_call(
        body,
        grid=(N // TCB,),
        in_specs=[pblk, blk, tspec, wspec, bspec],
        out_specs=blk,
        out_shape=jax.ShapeDtypeStruct((N, D), _f32),
    )(p4, hidp2, temp, W_out, b_out)


def kernel(x, edge_index, edge_w, W_in, b_in, W1, b1, W2, b2,
           We1, be1, We2, be2, W_out, b_out, temp):
    padi = jnp.zeros((EP - E,), _i32)
    src_r = jnp.concatenate([edge_index[0], padi]).reshape(NW, NCHUNK, CH)
    dst_r = jnp.concatenate([edge_index[1], padi]).reshape(NW, NCHUNK, CH)
    idxr = jnp.stack([src_r, dst_r], axis=2).reshape(
        NW, NCHUNK, 2, HS, CHH)
    aidxr = jnp.stack([src_r.reshape(NW, ANCH, ACH),
                       dst_r.reshape(NW, ANCH, ACH)], axis=2)
    padf = jnp.zeros((EP - E,), _f32)
    w_r = jnp.concatenate([edge_w, padf]).reshape(NW, NCHUNK, CH)

    h, lin1 = _tc_in(x, W_in, b_in, W1, b1)
    p1 = _spmm_sc(lin1, idxr, w_r)
    lin2, hidp = _tc_layer(p1, h, W2, b2, temp)
    p2 = _spmm_sc(lin2, idxr, w_r)
    g = _tc_extract(p2, hidp, temp, We1, be1, We2, be2)
    part = _attn_sc(g, aidxr)
    w2 = _tc_attnw(w_r.reshape(EP), part.reshape(EP, 48))
    w2_r = w2.reshape(NW, NCHUNK, CH)

    p3 = _spmm_sc(lin1, idxr, w2_r)
    lin2b, hidp2 = _tc_layer(p3, h, W2, b2, temp)
    p4 = _spmm_sc(lin2b, idxr, w2_r)
    return _tc_out(p4, hidp2, temp, W_out, b_out)


# speedup vs baseline: 2.5765x; 2.0579x over previous
import functools

import jax
import jax.numpy as jnp
from jax import lax
from jax.experimental import pallas as pl
from jax.experimental.pallas import tpu as pltpu
from jax.experimental.pallas import tpu_sc as plsc

N = 10000
E = 320000
D = 128
DH = D // 2
NC = 2
NS = 16
NW = NC * NS
EP = 327680
EPS = EP // NS
CH = 128
NCH = EPS // CH
RPT = 624
TCB = 1000

ACH = 64
ANCH = (EP // NW) // ACH
APACK = ACH * 48

_f32 = jnp.float32
_i32 = jnp.int32

_MESH = plsc.VectorSubcoreMesh(
    core_axis_name="c", subcore_axis_name="s", num_cores=NC, num_subcores=NS)


@functools.partial(
    pl.kernel,
    out_type=jax.ShapeDtypeStruct((NC, N, DH), _f32),
    mesh=_MESH,
    scratch_types=[
        pltpu.VMEM_SHARED((N, DH), _f32),
        pltpu.VMEM_SHARED((N, DH), _f32),
        pltpu.VMEM((3, 2, CH), _i32),
        pltpu.VMEM((3, CH), _f32),
        pltpu.VMEM((2, CH, DH), _f32),
        pltpu.SemaphoreType.DMA,
        pltpu.SemaphoreType.DMA,
        pltpu.SemaphoreType.DMA,
        pltpu.SemaphoreType.DMA,
    ],
)
def _spmm_sc(ta, tb, idxr, wr, out, stab, accum, idx_v, w_v, rows,
             isem, wsem, gsem, ssem):
    c = lax.axis_index("c")
    s = lax.axis_index("s")

    z16 = jnp.zeros((16,), _f32)

    @pl.loop(0, CH)
    def _zero_rows(i):
        for g in range(DH // 16):
            rows[0, i, pl.ds(g * 16, 16)] = z16

    base = s * RPT
    for k in range(RPT // CH):
        pltpu.sync_copy(rows.at[0], accum.at[pl.ds(base + k * CH, CH)])

    @pl.when(s == NS - 1)
    def _zero_tail_full():
        pltpu.sync_copy(rows.at[0], accum.at[pl.ds(base + 4 * CH, CH)])

    @pl.when(s != NS - 1)
    def _zero_tail_part():
        pltpu.sync_copy(rows.at[0, pl.ds(0, RPT - 4 * CH)],
                        accum.at[pl.ds(base + 4 * CH, RPT - 4 * CH)])

    @pl.when(c == 0)
    def _stage_a():
        pltpu.sync_copy(ta.at[pl.ds(base, RPT)], stab.at[pl.ds(base, RPT)])

        @pl.when(s == NS - 1)
        def _tail_a():
            pltpu.sync_copy(ta.at[pl.ds(N - 16, 16)],
                            stab.at[pl.ds(N - 16, 16)])

    @pl.when(c == 1)
    def _stage_b():
        pltpu.sync_copy(tb.at[pl.ds(base, RPT)], stab.at[pl.ds(base, RPT)])

        @pl.when(s == NS - 1)
        def _tail_b():
            pltpu.sync_copy(tb.at[pl.ds(N - 16, 16)],
                            stab.at[pl.ds(N - 16, 16)])

    plsc.subcore_barrier()

    pltpu.sync_copy(idxr.at[s, 0], idx_v.at[0])
    pltpu.sync_copy(wr.at[s, 0], w_v.at[0])
    pltpu.async_copy(stab.at[idx_v.at[0, 0]], rows.at[0], gsem)
    pltpu.async_copy(idxr.at[s, 1], idx_v.at[1], isem)
    pltpu.async_copy(wr.at[s, 1], w_v.at[1], wsem)

    @pl.loop(0, NCH)
    def _chunk(j):
        p = j % 2
        pn = (j + 1) % 2
        b0 = j % 3
        b1 = (j + 1) % 3
        b2 = (j + 2) % 3

        @pl.when(j >= 1)
        def _drain_prev_scatter():
            pltpu.make_async_copy(
                rows.at[pn], accum.at[idx_v.at[b2, 1]], ssem).wait()

        @pl.when(j + 1 < NCH)
        def _start_next_gather():
            pltpu.make_async_copy(idxr.at[s, j + 1], idx_v.at[b1],
                                  isem).wait()
            pltpu.async_copy(stab.at[idx_v.at[b1, 0]], rows.at[pn], gsem)

        @pl.when(j + 2 < NCH)
        def _start_next_idx():
            pltpu.async_copy(idxr.at[s, j + 2], idx_v.at[b2], isem)
            pltpu.async_copy(wr.at[s, j + 2], w_v.at[b2], wsem)

        pltpu.make_async_copy(stab.at[idx_v.at[b0, 0]], rows.at[p],
                              gsem).wait()

        @pl.when(j >= 1)
        def _wait_w():
            pltpu.make_async_copy(wr.at[s, j], w_v.at[b0], wsem).wait()

        @pl.loop(0, CH // 16)
        def _scale(t):
            wg = w_v[b0, pl.ds(t * 16, 16)]
            for k in range(16):
                e = t * 16 + k
                wv = jnp.full((16,), wg[k], _f32)
                for g in range(DH // 16):
                    rows[p, e, pl.ds(g * 16, 16)] = (
                        rows[p, e, pl.ds(g * 16, 16)] * wv)

        pltpu.async_copy(rows.at[p], accum.at[idx_v.at[b0, 1]], ssem,
                         add=True)

    pltpu.make_async_copy(
        rows.at[(NCH - 1) % 2],
        accum.at[idx_v.at[(NCH - 1) % 3, 1]], ssem).wait()

    plsc.subcore_barrier()
    for k in range(RPT // CH):
        pltpu.sync_copy(accum.at[pl.ds(base + k * CH, CH)],
                        out.at[c, pl.ds(base + k * CH, CH)])

    @pl.when(s == NS - 1)
    def _write_tail_full():
        pltpu.sync_copy(accum.at[pl.ds(base + 4 * CH, CH)],
                        out.at[c, pl.ds(base + 4 * CH, CH)])

    @pl.when(s != NS - 1)
    def _write_tail_part():
        pltpu.sync_copy(accum.at[pl.ds(base + 4 * CH, RPT - 4 * CH)],
                        out.at[c, pl.ds(base + 4 * CH, RPT - 4 * CH)])


@functools.partial(
    pl.kernel,
    out_type=jax.ShapeDtypeStruct((NW, ANCH, APACK), _f32),
    mesh=_MESH,
    scratch_types=[
        pltpu.VMEM_SHARED((N, D), _f32),
        pltpu.VMEM((3, 2, ACH), _i32),
        pltpu.VMEM((2, ACH, D), _f32),
        pltpu.VMEM((2, ACH, D), _f32),
        pltpu.VMEM((2, APACK), _f32),
        pltpu.SemaphoreType.DMA,
        pltpu.SemaphoreType.DMA,
        pltpu.SemaphoreType.DMA,
    ],
)
def _attn_sc(gtab, idxr, part_o, stab, idx_v, arows, brows, pall,
             isem, gsem, osem):
    c = lax.axis_index("c")
    s = lax.axis_index("s")
    wid = s * NC + c

    base = s * RPT
    pltpu.sync_copy(gtab.at[pl.ds(base, RPT)], stab.at[pl.ds(base, RPT)])

    @pl.when(s == NS - 1)
    def _stage_tail():
        pltpu.sync_copy(gtab.at[pl.ds(N - 16, 16)],
                        stab.at[pl.ds(N - 16, 16)])

    plsc.subcore_barrier()

    pltpu.sync_copy(idxr.at[wid, 0], idx_v.at[0])
    pltpu.async_copy(stab.at[idx_v.at[0, 0]], arows.at[0], gsem)
    pltpu.async_copy(stab.at[idx_v.at[0, 1]], brows.at[0], gsem)
    pltpu.async_copy(idxr.at[wid, 1], idx_v.at[1], isem)

    @pl.loop(0, ANCH)
    def _chunk(j):
        p = j % 2
        pn = (j + 1) % 2
        b0 = j % 3
        b1 = (j + 1) % 3
        b2 = (j + 2) % 3

        @pl.when(j >= 1)
        def _drain_prev_out():
            pltpu.make_async_copy(pall.at[pn], part_o.at[wid, j - 1],
                                  osem).wait()

        @pl.when(j + 1 < ANCH)
        def _start_next_gather():
            pltpu.make_async_copy(idxr.at[wid, j + 1], idx_v.at[b1],
                                  isem).wait()
            pltpu.async_copy(stab.at[idx_v.at[b1, 0]], arows.at[pn], gsem)
            pltpu.async_copy(stab.at[idx_v.at[b1, 1]], brows.at[pn], gsem)

        @pl.when(j + 2 < ANCH)
        def _start_next_idx():
            pltpu.async_copy(idxr.at[wid, j + 2], idx_v.at[b2], isem)

        pltpu.make_async_copy(stab.at[idx_v.at[b0, 0]], arows.at[p],
                              gsem).wait()
        pltpu.make_async_copy(stab.at[idx_v.at[b0, 1]], brows.at[p],
                              gsem).wait()

        @pl.loop(0, ACH, unroll=2)
        def _edge(e):
            a = arows[p, e, pl.ds(0, 16)]
            b = brows[p, e, pl.ds(0, 16)]
            pab = a * b
            paa = a * a
            pbb = b * b
            for g in range(1, D // 16):
                a = arows[p, e, pl.ds(g * 16, 16)]
                b = brows[p, e, pl.ds(g * 16, 16)]
                pab = pab + a * b
                paa = paa + a * a
                pbb = pbb + b * b
            pall[p, pl.ds(e * 48, 16)] = pab
            pall[p, pl.ds(e * 48 + 16, 16)] = paa
            pall[p, pl.ds(e * 48 + 32, 16)] = pbb

        pltpu.async_copy(pall.at[p], part_o.at[wid, j], osem)

    pltpu.make_async_copy(pall.at[(ANCH - 1) % 2],
                          part_o.at[wid, ANCH - 1], osem).wait()


def _dotT(a, w):
    return lax.dot_general(a, w, (((1,), (1,)), ((), ())),
                           preferred_element_type=_f32)


_blk = pl.BlockSpec((TCB, D), lambda i: (i, 0))
_hblk = pl.BlockSpec((TCB, DH), lambda i: (i, 0))
_pblk = pl.BlockSpec((NC, TCB, DH), lambda i: (0, i, 0))
_wspec = pl.BlockSpec((D, D), lambda i: (0, 0))
_bspec = pl.BlockSpec((D,), lambda i: (0,))
_tspec = pl.BlockSpec(memory_space=pltpu.SMEM)


def _tc_in(x, W_in, b_in, W1, b1):
    def body(x_r, wi_r, bi_r, w1_r, b1_r, h_r, la_r, lb_r):
        h = _dotT(x_r[...], wi_r[...]) + bi_r[...][None, :]
        h_r[...] = h
        lin1 = _dotT(h, w1_r[...]) + b1_r[...][None, :]
        la_r[...] = lin1[:, :DH]
        lb_r[...] = lin1[:, DH:]

    return pl.pallas_call(
        body,
        grid=(N // TCB,),
        in_specs=[_blk, _wspec, _bspec, _wspec, _bspec],
        out_specs=[_blk, _hblk, _hblk],
        out_shape=[jax.ShapeDtypeStruct((N, D), _f32),
                   jax.ShapeDtypeStruct((N, DH), _f32),
                   jax.ShapeDtypeStruct((N, DH), _f32)],
    )(x, W_in, b_in, W1, b1)


def _tc_layer(p, h, W2, b2, temp):
    def body(p_r, h_r, w2_r, b2_r, t_r, la_r, lb_r, hidp_r):
        cur = jnp.maximum(jnp.concatenate([p_r[0], p_r[1]], axis=1), 0.0)
        lin2 = _dotT(cur, w2_r[...]) + b2_r[...][None, :]
        la_r[...] = lin2[:, :DH]
        lb_r[...] = lin2[:, DH:]
        hidp_r[...] = h_r[...] * t_r[0] + cur * t_r[1]

    return pl.pallas_call(
        body,
        grid=(N // TCB,),
        in_specs=[_pblk, _blk, _wspec, _bspec, _tspec],
        out_specs=[_hblk, _hblk, _blk],
        out_shape=[jax.ShapeDtypeStruct((N, DH), _f32),
                   jax.ShapeDtypeStruct((N, DH), _f32),
                   jax.ShapeDtypeStruct((N, D), _f32)],
    )(p, h, W2, b2, temp)


def _tc_extract(p2, hidp, temp, We1, be1, We2, be2):
    def body(p_r, hidp_r, t_r, we1_r, be1_r, we2_r, be2_r, g_r):
        cur2 = jnp.maximum(jnp.concatenate([p_r[0], p_r[1]], axis=1), 0.0)
        hgnn = hidp_r[...] + cur2 * t_r[2]
        t1 = jnp.maximum(_dotT(hgnn, we1_r[...]) + be1_r[...][None, :], 0.0)
        g_r[...] = _dotT(t1, we2_r[...]) + be2_r[...][None, :]

    return pl.pallas_call(
        body,
        grid=(N // TCB,),
        in_specs=[_pblk, _blk, _tspec, _wspec, _bspec, _wspec, _bspec],
        out_specs=_blk,
        out_shape=jax.ShapeDtypeStruct((N, D), _f32),
    )(p2, hidp, temp, We1, be1, We2, be2)


def _tc_attnw(wf, part):
    TB = 8192

    def body(w_r, p_r, o_r):
        r = lax.broadcasted_iota(_i32, (48, 3), 0)
        cc = lax.broadcasted_iota(_i32, (48, 3), 1)
        sel = ((r // 16) == cc).astype(_f32)
        sums = jnp.dot(p_r[...], sel, preferred_element_type=_f32)
        num = sums[:, 0]
        den = jnp.maximum(jnp.sqrt(sums[:, 1] * sums[:, 2]), 1e-8)
        o_r[...] = w_r[...] * (num / den)

    return pl.pallas_call(
        body,
        grid=(EP // TB,),
        in_specs=[pl.BlockSpec((TB,), lambda i: (i,)),
                  pl.BlockSpec((TB, 48), lambda i: (i, 0))],
        out_specs=pl.BlockSpec((TB,), lambda i: (i,)),
        out_shape=jax.ShapeDtypeStruct((EP,), _f32),
    )(wf, part)


def _tc_out(p4, hidp2, temp, W_out, b_out):
    def body(p_r, hidp_r, t_r, wo_r, bo_r, o_r):
        cur = jnp.maximum(jnp.concatenate([p_r[0], p_r[1]], axis=1), 0.0)
        hgnn2 = hidp_r[...] + cur * t_r[2]
        o_r[...] = _dotT(hgnn2, wo_r[...]) + bo_r[...][None, :]

    return pl.pallas_call(
        body,
        grid=(N // TCB,),
        in_specs=[_pblk, _blk, _tspec, _wspec, _bspec],
        out_specs=_blk,
        out_shape=jax.ShapeDtypeStruct((N, D), _f32),
    )(p4, hidp2, temp, W_out, b_out)


def kernel(x, edge_index, edge_w, W_in, b_in, W1, b1, W2, b2,
           We1, be1, We2, be2, W_out, b_out, temp):
    padi = jnp.zeros((EP - E,), _i32)
    srcp = jnp.concatenate([edge_index[0], padi])
    dstp = jnp.concatenate([edge_index[1], padi])
    padf = jnp.zeros((EP - E,), _f32)
    wp = jnp.concatenate([edge_w, padf])

    idxr = jnp.stack([srcp.reshape(NS, NCH, CH),
                      dstp.reshape(NS, NCH, CH)], axis=2)
    wr = wp.reshape(NS, NCH, CH)
    aidxr = jnp.stack([srcp.reshape(NW, ANCH, ACH),
                       dstp.reshape(NW, ANCH, ACH)], axis=2)

    h, l1a, l1b = _tc_in(x, W_in, b_in, W1, b1)
    p1 = _spmm_sc(l1a, l1b, idxr, wr)
    l2a, l2b, hidp = _tc_layer(p1, h, W2, b2, temp)
    p2 = _spmm_sc(l2a, l2b, idxr, wr)
    g = _tc_extract(p2, hidp, temp, We1, be1, We2, be2)
    part = _attn_sc(g, aidxr)
    w2 = _tc_attnw(wp, part.reshape(EP, 48))
    w2r = w2.reshape(NS, NCH, CH)

    p3 = _spmm_sc(l1a, l1b, idxr, w2r)
    l2a2, l2b2, hidp2 = _tc_layer(p3, h, W2, b2, temp)
    p4 = _spmm_sc(l2a2, l2b2, idxr, w2r)
    return _tc_out(p4, hidp2, temp, W_out, b_out)
